# baseline (speedup 1.0000x reference)
import jax
import jax.numpy as jnp
from jax.experimental import pallas as pl
from jax.experimental.pallas import tpu as pltpu

_ROW_BLOCK = 400
_NBUF = 6


def _fused_heads_kernel(x_hbm, w_ref, b_ref, o1_ref, o2_ref, xbufs, sems):
    i = pl.program_id(0)
    nblocks = pl.num_programs(0)

    def start(blk, slot):
        pltpu.make_async_copy(
            x_hbm.at[pl.ds(blk * _ROW_BLOCK, _ROW_BLOCK), :],
            xbufs.at[slot], sems.at[slot]).start()

    @pl.when(i == 0)
    def _warmup():
        for k in range(_NBUF):
            start(k, k)

    slot = jax.lax.rem(i, _NBUF)
    pltpu.make_async_copy(
        x_hbm.at[pl.ds(i * _ROW_BLOCK, _ROW_BLOCK), :],
        xbufs.at[slot], sems.at[slot]).wait()
    acc = b_ref[...] + jnp.dot(xbufs[slot], w_ref[...],
                               preferred_element_type=jnp.float32)
    o1_ref[...] = acc[:, : o1_ref.shape[1]]
    o2_ref[...] = acc[:, 128 : 128 + o2_ref.shape[1]]

    nxt = i + _NBUF

    @pl.when(nxt < nblocks)
    def _prefetch():
        start(nxt, slot)


def kernel(x, W_cls, b_cls, W_box, b_box):
    if x.ndim > 2:
        x = x.reshape(x.shape[0], -1)
    R, D = x.shape
    n1 = W_cls.shape[1]
    n2 = W_box.shape[1]

    W = jnp.concatenate(
        [jnp.pad(W_cls, ((0, 0), (0, 128 - n1))),
         jnp.pad(W_box, ((0, 0), (0, 128 - n2)))], axis=1)
    b = jnp.concatenate(
        [jnp.pad(b_cls, (0, 128 - n1)), jnp.pad(b_box, (0, 128 - n2))]
    ).reshape(1, 256)

    o1, o2 = pl.pallas_call(
        _fused_heads_kernel,
        grid=(R // _ROW_BLOCK,),
        in_specs=[
            pl.BlockSpec(memory_space=pltpu.MemorySpace.HBM),
            pl.BlockSpec((D, 256), lambda i: (0, 0)),
            pl.BlockSpec((1, 256), lambda i: (0, 0)),
        ],
        out_specs=[
            pl.BlockSpec((_ROW_BLOCK, n1), lambda i: (i, 0)),
            pl.BlockSpec((_ROW_BLOCK, n2), lambda i: (i, 0)),
        ],
        out_shape=[
            jax.ShapeDtypeStruct((R, n1), jnp.float32),
            jax.ShapeDtypeStruct((R, n2), jnp.float32),
        ],
        scratch_shapes=[
            pltpu.VMEM((_NBUF, _ROW_BLOCK, D), jnp.float32),
            pltpu.SemaphoreType.DMA((_NBUF,)),
        ],
    )(x, W, b)

    return o1, o2
